# baseline (speedup 1.0000x reference)
import functools

import jax
import jax.numpy as jnp
from jax import lax
from jax.experimental import pallas as pl
from jax.experimental.pallas import tpu as pltpu
from jax.experimental.pallas import tpu_sc as plsc

D_MODEL = 2048
N_EXP = 8
HIDDEN = 8192
TOP_K = 2

ROUTER_BT = 512
GEMM_BT = 768
GEMM_HT = 512
SC_CHUNK = 16


def _router_kernel(x_ref, wr_ref, br_ref, idx_ref, wbc_ref, rank_ref,
                   counts_ref, stats_ref, lbl_ref, *, n_tiles, total_t):
    i = pl.program_id(0)
    logits = jnp.dot(x_ref[...], wr_ref[...], preferred_element_type=jnp.float32)
    logits = logits + br_ref[...]
    m = jnp.max(logits, axis=-1, keepdims=True)
    ex = jnp.exp(logits - m)
    probs = ex / jnp.sum(ex, axis=-1, keepdims=True)

    lane = jax.lax.broadcasted_iota(jnp.int32, probs.shape, 1)
    p1 = jnp.max(probs, axis=-1, keepdims=True)
    i1 = jnp.argmax(probs, axis=-1)[:, None]
    oh1 = lane == i1
    probs_m = jnp.where(oh1, -jnp.inf, probs)
    p2 = jnp.max(probs_m, axis=-1, keepdims=True)
    i2 = jnp.argmax(probs_m, axis=-1)[:, None]
    oh2 = lane == i2
    denom = p1 + p2

    idx_ref[...] = jnp.concatenate([i1.T, i2.T], axis=0)
    bt = p1.shape[0]
    wbc_ref[...] = jnp.concatenate(
        [jnp.broadcast_to(p1 / denom, (bt, 16)),
         jnp.broadcast_to(p2 / denom, (bt, 16))], axis=1)

    oh1f = oh1.astype(jnp.float32)
    oh2f = oh2.astype(jnp.float32)
    a = oh1f + oh2f
    s = a
    k = 1
    while k < s.shape[0]:
        shifted = jnp.concatenate([jnp.zeros((k, s.shape[1]), s.dtype), s[:-k]],
                                  axis=0)
        s = s + shifted
        k *= 2
    excl = s - a

    @pl.when(i == 0)
    def _():
        counts_ref[...] = jnp.zeros_like(counts_ref)

    base = counts_ref[...]
    r1 = jnp.sum((base + excl) * oh1f, axis=1, keepdims=True)
    r2 = jnp.sum((base + excl) * oh2f, axis=1, keepdims=True)
    rank_ref[...] = jnp.concatenate([r1.T, r2.T], axis=0).astype(jnp.int32)
    counts_ref[...] = base + jnp.sum(a, axis=0, keepdims=True)

    psum = jnp.sum(probs, axis=0, keepdims=True)

    @pl.when(i == 0)
    def _():
        stats_ref[...] = psum

    @pl.when(i > 0)
    def _():
        stats_ref[...] = stats_ref[...] + psum

    @pl.when(i == n_tiles - 1)
    def _():
        pm = stats_ref[...] / total_t
        lbl_ref[...] = jnp.sum(pm * jnp.log(pm)).reshape(1, 1) * N_EXP


def _gemm_kernel(meta_ref, x_ref, w1_ref, b1_ref, w2_ref, b2_ref, y_ref,
                 xb_ref, ha_ref):
    b = pl.program_id(0)
    h = pl.program_id(1)
    valid = meta_ref[2, b]

    @pl.when(valid == 1)
    def _():
        @pl.when(h == 0)
        def _():
            xb_ref[...] = x_ref[...].astype(jnp.bfloat16)

        n_ht = b1_ref.shape[1]
        oh_h = (jax.lax.broadcasted_iota(jnp.int32, (1, n_ht), 1) == h).astype(jnp.float32)
        b1row = jnp.dot(oh_h, b1_ref[0], preferred_element_type=jnp.float32)
        hid = jnp.dot(xb_ref[...], w1_ref[0].astype(jnp.bfloat16),
                      preferred_element_type=jnp.float32)
        hid = hid + b1row
        ha_ref[...] = (0.5 * hid * (1.0 + jax.lax.erf(hid * 0.7071067811865476))
                       ).astype(jnp.bfloat16)
        part = jnp.dot(ha_ref[...], w2_ref[0].astype(jnp.bfloat16),
                       preferred_element_type=jnp.float32)

        @pl.when(h == 0)
        def _():
            y_ref[...] = part + b2_ref[0]

        @pl.when(h > 0)
        def _():
            y_ref[...] = y_ref[...] + part


def _positions(i_ref, r_ref, st_ref, sl):
    e = i_ref[sl]
    p = r_ref[sl]
    for ex in range(N_EXP):
        p = p + jnp.where(e == ex, st_ref[ex], 0)
    return p


def _sc_scatter_call(x_flat, idx_t, rank_t, starts_b, np_pad):
    t, d = x_flat.shape
    nw = 32
    tpw = t // nw
    nch = tpw // SC_CHUNK
    mesh = plsc.VectorSubcoreMesh(core_axis_name="c", subcore_axis_name="s")

    @functools.partial(
        pl.kernel, mesh=mesh,
        out_type=jax.ShapeDtypeStruct((np_pad, d), jnp.float32),
        scratch_types=[
            pltpu.VMEM((tpw,), jnp.int32),
            pltpu.VMEM((tpw,), jnp.int32),
            pltpu.VMEM((tpw,), jnp.int32),
            pltpu.VMEM((tpw,), jnp.int32),
            pltpu.VMEM((N_EXP, 16), jnp.int32),
            pltpu.VMEM((SC_CHUNK,), jnp.int32),
            pltpu.VMEM((SC_CHUNK,), jnp.int32),
            pltpu.VMEM((SC_CHUNK, d), jnp.float32),
            pltpu.SemaphoreType.DMA,
        ],
    )
    def scatter(x_hbm, it_hbm, rt_hbm, st_hbm, xs_hbm,
                i0a, i1a, r0a, r1a, st_v, p0v, p1v, rows_v, sem):
        wid = lax.axis_index("s") * 2 + lax.axis_index("c")
        base = wid * tpw
        pltpu.sync_copy(st_hbm, st_v)
        pltpu.sync_copy(it_hbm.at[0, pl.ds(base, tpw)], i0a)
        pltpu.sync_copy(it_hbm.at[1, pl.ds(base, tpw)], i1a)
        pltpu.sync_copy(rt_hbm.at[0, pl.ds(base, tpw)], r0a)
        pltpu.sync_copy(rt_hbm.at[1, pl.ds(base, tpw)], r1a)
        for c in range(nch):
            off = base + c * SC_CHUNK
            sl = pl.ds(c * SC_CHUNK, SC_CHUNK)
            pltpu.sync_copy(x_hbm.at[pl.ds(off, SC_CHUNK)], rows_v)
            p0v[...] = _positions(i0a, r0a, st_v, sl)
            p1v[...] = _positions(i1a, r1a, st_v, sl)
            cp0 = pltpu.async_copy(rows_v, xs_hbm.at[p0v], sem)
            cp1 = pltpu.async_copy(rows_v, xs_hbm.at[p1v], sem)
            cp0.wait()
            cp1.wait()

    return scatter(x_flat, idx_t, rank_t, starts_b)


def _sc_combine_call(y, idx_t, rank_t, starts_b, wbc, t):
    d = y.shape[1]
    nw = 32
    tpw = t // nw
    nch = tpw // SC_CHUNK
    mesh = plsc.VectorSubcoreMesh(core_axis_name="c", subcore_axis_name="s")

    cc = 8
    ncc = tpw // cc

    @functools.partial(
        pl.kernel, mesh=mesh,
        out_type=jax.ShapeDtypeStruct((t, d), jnp.float32),
        scratch_types=[
            pltpu.VMEM((tpw,), jnp.int32),
            pltpu.VMEM((tpw,), jnp.int32),
            pltpu.VMEM((tpw,), jnp.int32),
            pltpu.VMEM((tpw,), jnp.int32),
            pltpu.VMEM((N_EXP, 16), jnp.int32),
            pltpu.VMEM((tpw,), jnp.int32),
            pltpu.VMEM((tpw,), jnp.int32),
            pltpu.VMEM((tpw, 32), jnp.float32),
            pltpu.VMEM((2, 2 * cc, d), jnp.float32),
            pltpu.VMEM((cc, d), jnp.float32),
            pltpu.SemaphoreType.DMA,
            pltpu.SemaphoreType.DMA,
        ],
    )
    def combine(y_hbm, it_hbm, rt_hbm, st_hbm, wbc_hbm, out_hbm,
                i0a, i1a, r0a, r1a, st_v, p0a, p1a, wb_v, r_v, o_v,
                sem0, sem1):
        wid = lax.axis_index("s") * 2 + lax.axis_index("c")
        base = wid * tpw
        pltpu.sync_copy(st_hbm, st_v)
        pltpu.sync_copy(it_hbm.at[0, pl.ds(base, tpw)], i0a)
        pltpu.sync_copy(it_hbm.at[1, pl.ds(base, tpw)], i1a)
        pltpu.sync_copy(rt_hbm.at[0, pl.ds(base, tpw)], r0a)
        pltpu.sync_copy(rt_hbm.at[1, pl.ds(base, tpw)], r1a)
        pltpu.sync_copy(wbc_hbm.at[pl.ds(base, tpw)], wb_v)
        for c in range(tpw // 16):
            sl = pl.ds(c * 16, 16)
            p0a[sl] = _positions(i0a, r0a, st_v, sl)
            p1a[sl] = _positions(i1a, r1a, st_v, sl)

        sems = [sem0, sem1]

        def gathers(c, buf):
            s = sems[buf]
            g0 = pltpu.async_copy(y_hbm.at[p0a.at[pl.ds(c * cc, cc)]],
                                  r_v.at[buf, pl.ds(0, cc)], s)
            g1 = pltpu.async_copy(y_hbm.at[p1a.at[pl.ds(c * cc, cc)]],
                                  r_v.at[buf, pl.ds(cc, cc)], s)
            return g0, g1

        pend = gathers(0, 0)
        for c in range(ncc):
            buf = c % 2
            pend[0].wait()
            pend[1].wait()
            if c + 1 < ncc:
                pend = gathers(c + 1, 1 - buf)
            for tk in range(cc):
                wv0 = wb_v[c * cc + tk, pl.ds(0, 16)]
                wv1 = wb_v[c * cc + tk, pl.ds(16, 16)]

                def d_body(db, _, tk=tk, wv0=wv0, wv1=wv1, buf=buf):
                    dsl = pl.ds(db * 16, 16)
                    o_v[tk, dsl] = (wv0 * r_v[buf, tk, dsl]
                                    + wv1 * r_v[buf, tk + cc, dsl])
                    return 0

                lax.fori_loop(0, d // 16, d_body, 0)
            pltpu.sync_copy(o_v, out_hbm.at[pl.ds(base + c * cc, cc)])

    return combine(y, idx_t, rank_t, starts_b, wbc)


def _moe(x, Wr, br, W1, b1, W2, b2, *, interpret=False):
    B, S, D = x.shape
    T = B * S
    x_flat = x.reshape(T, D)

    n_rt = T // ROUTER_BT
    idx_t, wbc, rank_t, counts_f, stats, lbl = pl.pallas_call(
        functools.partial(_router_kernel, n_tiles=n_rt, total_t=float(T)),
        grid=(n_rt,),
        in_specs=[
            pl.BlockSpec((ROUTER_BT, D), lambda i: (i, 0)),
            pl.BlockSpec((D, N_EXP), lambda i: (0, 0)),
            pl.BlockSpec((1, N_EXP), lambda i: (0, 0)),
        ],
        out_specs=[
            pl.BlockSpec((TOP_K, ROUTER_BT), lambda i: (0, i)),
            pl.BlockSpec((ROUTER_BT, 32), lambda i: (i, 0)),
            pl.BlockSpec((TOP_K, ROUTER_BT), lambda i: (0, i)),
            pl.BlockSpec((1, N_EXP), lambda i: (0, 0)),
            pl.BlockSpec((1, N_EXP), lambda i: (0, 0)),
            pl.BlockSpec((1, 1), lambda i: (0, 0)),
        ],
        out_shape=[
            jax.ShapeDtypeStruct((TOP_K, T), jnp.int32),
            jax.ShapeDtypeStruct((T, 32), jnp.float32),
            jax.ShapeDtypeStruct((TOP_K, T), jnp.int32),
            jax.ShapeDtypeStruct((1, N_EXP), jnp.float32),
            jax.ShapeDtypeStruct((1, N_EXP), jnp.float32),
            jax.ShapeDtypeStruct((1, 1), jnp.float32),
        ],
        interpret=interpret,
    )(x_flat, Wr, br.reshape(1, N_EXP))

    nb = -((-T * TOP_K) // GEMM_BT) + N_EXP - 1
    np_pad = nb * GEMM_BT
    counts = counts_f[0].astype(jnp.int32)
    nb_e = (counts + GEMM_BT - 1) // GEMM_BT
    cumb = jnp.cumsum(nb_e)
    total_b = cumb[N_EXP - 1]
    starts_pad = (cumb - nb_e) * GEMM_BT
    starts_b = jnp.broadcast_to(starts_pad[:, None], (N_EXP, 16)).astype(jnp.int32)

    barange = jnp.arange(nb, dtype=jnp.int32)
    bsrc = jnp.minimum(barange, total_b - 1)
    bexp = jnp.sum((bsrc[:, None] >= cumb[None, :]).astype(jnp.int32), axis=1)
    bvalid = (barange < total_b).astype(jnp.int32)
    meta = jnp.stack([bsrc, bexp, bvalid])

    xs = _sc_scatter_call(x_flat, idx_t, rank_t, starts_b, np_pad)

    n_ht = HIDDEN // GEMM_HT
    grid_spec = pltpu.PrefetchScalarGridSpec(
        num_scalar_prefetch=1,
        grid=(nb, n_ht),
        in_specs=[
            pl.BlockSpec((GEMM_BT, D), lambda b, h, m: (m[0, b], 0)),
            pl.BlockSpec((1, D, GEMM_HT), lambda b, h, m: (m[1, b], 0, h)),
            pl.BlockSpec((1, n_ht, GEMM_HT), lambda b, h, m: (m[1, b], 0, 0)),
            pl.BlockSpec((1, GEMM_HT, D), lambda b, h, m: (m[1, b], h, 0)),
            pl.BlockSpec((1, 1, D), lambda b, h, m: (m[1, b], 0, 0)),
        ],
        out_specs=pl.BlockSpec((GEMM_BT, D), lambda b, h, m: (m[0, b], 0)),
        scratch_shapes=[pltpu.VMEM((GEMM_BT, D), jnp.bfloat16),
                        pltpu.VMEM((GEMM_BT, GEMM_HT), jnp.bfloat16)],
    )
    y = pl.pallas_call(
        _gemm_kernel,
        grid_spec=grid_spec,
        out_shape=jax.ShapeDtypeStruct((np_pad, D), jnp.float32),
        compiler_params=pltpu.CompilerParams(
            dimension_semantics=("arbitrary", "arbitrary"),
        ),
        interpret=interpret,
    )(meta, xs, W1, b1.reshape(N_EXP, n_ht, GEMM_HT), W2,
      b2.reshape(N_EXP, 1, D))

    out = _sc_combine_call(y, idx_t, rank_t, starts_b, wbc, T)
    return out.reshape(B, S, D), lbl[0, 0]


def kernel(x, Wr, br, W1, b1, W2, b2):
    return _moe(x, Wr, br, W1, b1, W2, b2)


# speedup vs baseline: 1.1296x; 1.1296x over previous
import functools

import jax
import jax.numpy as jnp
from jax import lax
from jax.experimental import pallas as pl
from jax.experimental.pallas import tpu as pltpu
from jax.experimental.pallas import tpu_sc as plsc

D_MODEL = 2048
N_EXP = 8
HIDDEN = 8192
TOP_K = 2

ROUTER_BT = 512
GEMM_BT = 512
GEMM_HT = 1024
SC_CHUNK = 16


def _router_kernel(x_ref, wr_ref, br_ref, idx_ref, wbc_ref, rank_ref,
                   counts_ref, stats_ref, lbl_ref, *, n_tiles, total_t):
    i = pl.program_id(0)
    logits = jnp.dot(x_ref[...], wr_ref[...], preferred_element_type=jnp.float32)
    logits = logits + br_ref[...]
    m = jnp.max(logits, axis=-1, keepdims=True)
    ex = jnp.exp(logits - m)
    probs = ex / jnp.sum(ex, axis=-1, keepdims=True)

    lane = jax.lax.broadcasted_iota(jnp.int32, probs.shape, 1)
    p1 = jnp.max(probs, axis=-1, keepdims=True)
    i1 = jnp.argmax(probs, axis=-1)[:, None]
    oh1 = lane == i1
    probs_m = jnp.where(oh1, -jnp.inf, probs)
    p2 = jnp.max(probs_m, axis=-1, keepdims=True)
    i2 = jnp.argmax(probs_m, axis=-1)[:, None]
    oh2 = lane == i2
    denom = p1 + p2

    idx_ref[...] = jnp.concatenate([i1.T, i2.T], axis=0)
    bt = p1.shape[0]
    wbc_ref[...] = jnp.concatenate(
        [jnp.broadcast_to(p1 / denom, (bt, 16)),
         jnp.broadcast_to(p2 / denom, (bt, 16))], axis=1)

    oh1f = oh1.astype(jnp.float32)
    oh2f = oh2.astype(jnp.float32)
    a = oh1f + oh2f
    s = a
    k = 1
    while k < s.shape[0]:
        shifted = jnp.concatenate([jnp.zeros((k, s.shape[1]), s.dtype), s[:-k]],
                                  axis=0)
        s = s + shifted
        k *= 2
    excl = s - a

    @pl.when(i == 0)
    def _():
        counts_ref[...] = jnp.zeros_like(counts_ref)

    base = counts_ref[...]
    r1 = jnp.sum((base + excl) * oh1f, axis=1, keepdims=True)
    r2 = jnp.sum((base + excl) * oh2f, axis=1, keepdims=True)
    rank_ref[...] = jnp.concatenate([r1.T, r2.T], axis=0).astype(jnp.int32)
    counts_ref[...] = base + jnp.sum(a, axis=0, keepdims=True)

    psum = jnp.sum(probs, axis=0, keepdims=True)

    @pl.when(i == 0)
    def _():
        stats_ref[...] = psum

    @pl.when(i > 0)
    def _():
        stats_ref[...] = stats_ref[...] + psum

    @pl.when(i == n_tiles - 1)
    def _():
        pm = stats_ref[...] / total_t
        lbl_ref[...] = jnp.sum(pm * jnp.log(pm)).reshape(1, 1) * N_EXP


def _gemm_kernel(meta_ref, x_ref, w1_ref, b1_ref, w2_ref, b2_ref, y_ref,
                 xb_ref, ha_ref):
    b = pl.program_id(0)
    h = pl.program_id(1)
    valid = meta_ref[2, b]

    @pl.when(valid == 1)
    def _():
        @pl.when(h == 0)
        def _():
            xb_ref[...] = x_ref[...].astype(jnp.bfloat16)

        n_ht = b1_ref.shape[1]
        oh_h = (jax.lax.broadcasted_iota(jnp.int32, (1, n_ht), 1) == h).astype(jnp.float32)
        b1row = jnp.dot(oh_h, b1_ref[0], preferred_element_type=jnp.float32)
        hid = jnp.dot(xb_ref[...], w1_ref[0].astype(jnp.bfloat16),
                      preferred_element_type=jnp.float32)
        hid = hid + b1row
        ha_ref[...] = (0.5 * hid * (1.0 + jax.lax.erf(hid * 0.7071067811865476))
                       ).astype(jnp.bfloat16)
        part = jnp.dot(ha_ref[...], w2_ref[0].astype(jnp.bfloat16),
                       preferred_element_type=jnp.float32)

        @pl.when(h == 0)
        def _():
            y_ref[...] = part + b2_ref[0]

        @pl.when(h > 0)
        def _():
            y_ref[...] = y_ref[...] + part


def _positions(i_ref, r_ref, st_ref, sl):
    e = i_ref[sl]
    p = r_ref[sl]
    for ex in range(N_EXP):
        p = p + jnp.where(e == ex, st_ref[ex], 0)
    return p


def _sc_scatter_call(x_flat, idx_t, rank_t, starts_b, np_pad):
    t, d = x_flat.shape
    nw = 32
    tpw = t // nw
    nch = tpw // SC_CHUNK
    mesh = plsc.VectorSubcoreMesh(core_axis_name="c", subcore_axis_name="s")

    @functools.partial(
        pl.kernel, mesh=mesh,
        out_type=jax.ShapeDtypeStruct((np_pad, d), jnp.float32),
        scratch_types=[
            pltpu.VMEM((tpw,), jnp.int32),
            pltpu.VMEM((tpw,), jnp.int32),
            pltpu.VMEM((tpw,), jnp.int32),
            pltpu.VMEM((tpw,), jnp.int32),
            pltpu.VMEM((N_EXP, 16), jnp.int32),
            pltpu.VMEM((SC_CHUNK,), jnp.int32),
            pltpu.VMEM((SC_CHUNK,), jnp.int32),
            pltpu.VMEM((SC_CHUNK, d), jnp.float32),
            pltpu.SemaphoreType.DMA,
        ],
    )
    def scatter(x_hbm, it_hbm, rt_hbm, st_hbm, xs_hbm,
                i0a, i1a, r0a, r1a, st_v, p0v, p1v, rows_v, sem):
        wid = lax.axis_index("s") * 2 + lax.axis_index("c")
        base = wid * tpw
        pltpu.sync_copy(st_hbm, st_v)
        pltpu.sync_copy(it_hbm.at[0, pl.ds(base, tpw)], i0a)
        pltpu.sync_copy(it_hbm.at[1, pl.ds(base, tpw)], i1a)
        pltpu.sync_copy(rt_hbm.at[0, pl.ds(base, tpw)], r0a)
        pltpu.sync_copy(rt_hbm.at[1, pl.ds(base, tpw)], r1a)
        for c in range(nch):
            off = base + c * SC_CHUNK
            sl = pl.ds(c * SC_CHUNK, SC_CHUNK)
            pltpu.sync_copy(x_hbm.at[pl.ds(off, SC_CHUNK)], rows_v)
            p0v[...] = _positions(i0a, r0a, st_v, sl)
            p1v[...] = _positions(i1a, r1a, st_v, sl)
            cp0 = pltpu.async_copy(rows_v, xs_hbm.at[p0v], sem)
            cp1 = pltpu.async_copy(rows_v, xs_hbm.at[p1v], sem)
            cp0.wait()
            cp1.wait()

    return scatter(x_flat, idx_t, rank_t, starts_b)


def _sc_combine_call(y, idx_t, rank_t, starts_b, wbc, t):
    d = y.shape[1]
    nw = 32
    tpw = t // nw
    nch = tpw // SC_CHUNK
    mesh = plsc.VectorSubcoreMesh(core_axis_name="c", subcore_axis_name="s")

    cc = 8
    ncc = tpw // cc

    @functools.partial(
        pl.kernel, mesh=mesh,
        out_type=jax.ShapeDtypeStruct((t, d), jnp.float32),
        scratch_types=[
            pltpu.VMEM((tpw,), jnp.int32),
            pltpu.VMEM((tpw,), jnp.int32),
            pltpu.VMEM((tpw,), jnp.int32),
            pltpu.VMEM((tpw,), jnp.int32),
            pltpu.VMEM((N_EXP, 16), jnp.int32),
            pltpu.VMEM((tpw,), jnp.int32),
            pltpu.VMEM((tpw,), jnp.int32),
            pltpu.VMEM((tpw, 32), jnp.float32),
            pltpu.VMEM((2, 2 * cc, d), jnp.float32),
            pltpu.VMEM((cc, d), jnp.float32),
            pltpu.SemaphoreType.DMA,
            pltpu.SemaphoreType.DMA,
        ],
    )
    def combine(y_hbm, it_hbm, rt_hbm, st_hbm, wbc_hbm, out_hbm,
                i0a, i1a, r0a, r1a, st_v, p0a, p1a, wb_v, r_v, o_v,
                sem0, sem1):
        wid = lax.axis_index("s") * 2 + lax.axis_index("c")
        base = wid * tpw
        pltpu.sync_copy(st_hbm, st_v)
        pltpu.sync_copy(it_hbm.at[0, pl.ds(base, tpw)], i0a)
        pltpu.sync_copy(it_hbm.at[1, pl.ds(base, tpw)], i1a)
        pltpu.sync_copy(rt_hbm.at[0, pl.ds(base, tpw)], r0a)
        pltpu.sync_copy(rt_hbm.at[1, pl.ds(base, tpw)], r1a)
        pltpu.sync_copy(wbc_hbm.at[pl.ds(base, tpw)], wb_v)
        for c in range(tpw // 16):
            sl = pl.ds(c * 16, 16)
            p0a[sl] = _positions(i0a, r0a, st_v, sl)
            p1a[sl] = _positions(i1a, r1a, st_v, sl)

        sems = [sem0, sem1]

        def gathers(c, buf):
            s = sems[buf]
            g0 = pltpu.async_copy(y_hbm.at[p0a.at[pl.ds(c * cc, cc)]],
                                  r_v.at[buf, pl.ds(0, cc)], s)
            g1 = pltpu.async_copy(y_hbm.at[p1a.at[pl.ds(c * cc, cc)]],
                                  r_v.at[buf, pl.ds(cc, cc)], s)
            return g0, g1

        pend = gathers(0, 0)
        for c in range(ncc):
            buf = c % 2
            pend[0].wait()
            pend[1].wait()
            if c + 1 < ncc:
                pend = gathers(c + 1, 1 - buf)
            for tk in range(cc):
                wv0 = wb_v[c * cc + tk, pl.ds(0, 16)]
                wv1 = wb_v[c * cc + tk, pl.ds(16, 16)]

                def d_body(db, _, tk=tk, wv0=wv0, wv1=wv1, buf=buf):
                    dsl = pl.ds(db * 16, 16)
                    o_v[tk, dsl] = (wv0 * r_v[buf, tk, dsl]
                                    + wv1 * r_v[buf, tk + cc, dsl])
                    return 0

                lax.fori_loop(0, d // 16, d_body, 0)
            pltpu.sync_copy(o_v, out_hbm.at[pl.ds(base + c * cc, cc)])

    return combine(y, idx_t, rank_t, starts_b, wbc)


def _moe(x, Wr, br, W1, b1, W2, b2, *, interpret=False):
    B, S, D = x.shape
    T = B * S
    x_flat = x.reshape(T, D)

    n_rt = T // ROUTER_BT
    idx_t, wbc, rank_t, counts_f, stats, lbl = pl.pallas_call(
        functools.partial(_router_kernel, n_tiles=n_rt, total_t=float(T)),
        grid=(n_rt,),
        in_specs=[
            pl.BlockSpec((ROUTER_BT, D), lambda i: (i, 0)),
            pl.BlockSpec((D, N_EXP), lambda i: (0, 0)),
            pl.BlockSpec((1, N_EXP), lambda i: (0, 0)),
        ],
        out_specs=[
            pl.BlockSpec((TOP_K, ROUTER_BT), lambda i: (0, i)),
            pl.BlockSpec((ROUTER_BT, 32), lambda i: (i, 0)),
            pl.BlockSpec((TOP_K, ROUTER_BT), lambda i: (0, i)),
            pl.BlockSpec((1, N_EXP), lambda i: (0, 0)),
            pl.BlockSpec((1, N_EXP), lambda i: (0, 0)),
            pl.BlockSpec((1, 1), lambda i: (0, 0)),
        ],
        out_shape=[
            jax.ShapeDtypeStruct((TOP_K, T), jnp.int32),
            jax.ShapeDtypeStruct((T, 32), jnp.float32),
            jax.ShapeDtypeStruct((TOP_K, T), jnp.int32),
            jax.ShapeDtypeStruct((1, N_EXP), jnp.float32),
            jax.ShapeDtypeStruct((1, N_EXP), jnp.float32),
            jax.ShapeDtypeStruct((1, 1), jnp.float32),
        ],
        interpret=interpret,
    )(x_flat, Wr, br.reshape(1, N_EXP))

    nb = -((-T * TOP_K) // GEMM_BT) + N_EXP - 1
    np_pad = nb * GEMM_BT
    counts = counts_f[0].astype(jnp.int32)
    nb_e = (counts + GEMM_BT - 1) // GEMM_BT
    cumb = jnp.cumsum(nb_e)
    total_b = cumb[N_EXP - 1]
    starts_pad = (cumb - nb_e) * GEMM_BT
    starts_b = jnp.broadcast_to(starts_pad[:, None], (N_EXP, 16)).astype(jnp.int32)

    barange = jnp.arange(nb, dtype=jnp.int32)
    bsrc = jnp.minimum(barange, total_b - 1)
    bexp = jnp.sum((bsrc[:, None] >= cumb[None, :]).astype(jnp.int32), axis=1)
    bvalid = (barange < total_b).astype(jnp.int32)
    meta = jnp.stack([bsrc, bexp, bvalid])

    xs = _sc_scatter_call(x_flat, idx_t, rank_t, starts_b, np_pad)

    n_ht = HIDDEN // GEMM_HT
    grid_spec = pltpu.PrefetchScalarGridSpec(
        num_scalar_prefetch=1,
        grid=(nb, n_ht),
        in_specs=[
            pl.BlockSpec((GEMM_BT, D), lambda b, h, m: (m[0, b], 0)),
            pl.BlockSpec((1, D, GEMM_HT), lambda b, h, m: (m[1, b], 0, h)),
            pl.BlockSpec((1, n_ht, GEMM_HT), lambda b, h, m: (m[1, b], 0, 0)),
            pl.BlockSpec((1, GEMM_HT, D), lambda b, h, m: (m[1, b], h, 0)),
            pl.BlockSpec((1, 1, D), lambda b, h, m: (m[1, b], 0, 0)),
        ],
        out_specs=pl.BlockSpec((GEMM_BT, D), lambda b, h, m: (m[0, b], 0)),
        scratch_shapes=[pltpu.VMEM((GEMM_BT, D), jnp.bfloat16),
                        pltpu.VMEM((GEMM_BT, GEMM_HT), jnp.bfloat16)],
    )
    y = pl.pallas_call(
        _gemm_kernel,
        grid_spec=grid_spec,
        out_shape=jax.ShapeDtypeStruct((np_pad, D), jnp.float32),
        compiler_params=pltpu.CompilerParams(
            dimension_semantics=("arbitrary", "arbitrary"),
        ),
        interpret=interpret,
    )(meta, xs, W1, b1.reshape(N_EXP, n_ht, GEMM_HT), W2,
      b2.reshape(N_EXP, 1, D))

    out = _sc_combine_call(y, idx_t, rank_t, starts_b, wbc, T)
    return out.reshape(B, S, D), lbl[0, 0]


def kernel(x, Wr, br, W1, b1, W2, b2):
    return _moe(x, Wr, br, W1, b1, W2, b2)
